# Initial kernel scaffold; baseline (speedup 1.0000x reference)
import jax
import jax.numpy as jnp
from jax.experimental import pallas as pl


def kernel(seq_lengths, embeddings):
    raise NotImplementedError("write your pallas kernel here")


# speedup vs baseline: 1.3191x; 1.3191x over previous
import jax
import jax.numpy as jnp
from jax import lax
from jax.experimental import pallas as pl
from jax.experimental.pallas import tpu as pltpu
from jax.experimental.pallas import tpu_sc as plsc

CONTEXT = 2048
EMB = 1024
BATCH = 8
NWORK = 32
ROWS_PER_W = (BATCH * CONTEXT) // NWORK
K = 32
NCHUNK = ROWS_PER_W // K
NBUF = 3


def _body(seq_hbm, table_hbm, out_hbm,
          seq_v, idx_v, buf0, buf1, buf2,
          gs0, gs1, gs2, ss0, ss1, ss2):
    cid = lax.axis_index("c")
    sid = lax.axis_index("s")
    w = sid * 2 + cid
    b = w // 4
    qtr = lax.rem(w, 4)
    base = qtr * ROWS_PER_W
    obase = b * CONTEXT + base

    pltpu.sync_copy(seq_hbm.at[w], seq_v)
    s_vec = seq_v[...]

    lane = lax.iota(jnp.int32, 16)
    for t in range(ROWS_PER_W // 16):
        v = lane + (base + 16 * t) + s_vec
        idx_v[pl.ds(16 * t, 16)] = v & (CONTEXT - 1)

    bufs = (buf0, buf1, buf2)
    gsems = (gs0, gs1, gs2)
    ssems = (ss0, ss1, ss2)
    gd = [None] * NCHUNK
    sd = [None] * NCHUNK

    def fire_gather(i):
        slot = i % NBUF
        gd[i] = pltpu.async_copy(
            table_hbm.at[idx_v.at[pl.ds(i * K, K)]], bufs[slot], gsems[slot])

    for i in range(NBUF):
        fire_gather(i)
    for i in range(NCHUNK):
        slot = i % NBUF
        if i >= 1 and i + NBUF - 1 < NCHUNK:
            sd[i - 1].wait()
            fire_gather(i + NBUF - 1)
        gd[i].wait()
        sd[i] = pltpu.async_copy(
            bufs[slot], out_hbm.at[pl.ds(obase + i * K, K)], ssems[slot])
    for i in range(NCHUNK - NBUF, NCHUNK):
        sd[i].wait()


_roll_cache = []


def _get_roll():
    if not _roll_cache:
        mesh = plsc.VectorSubcoreMesh(core_axis_name="c", subcore_axis_name="s",
                                      num_cores=2, num_subcores=16)
        _roll_cache.append(pl.kernel(
            _body,
            out_type=jax.ShapeDtypeStruct((BATCH * CONTEXT, EMB), jnp.float32),
            mesh=mesh,
            scratch_types=[
                pltpu.VMEM((16,), jnp.int32),
                pltpu.VMEM((ROWS_PER_W,), jnp.int32),
                pltpu.VMEM((K, EMB), jnp.float32),
                pltpu.VMEM((K, EMB), jnp.float32),
                pltpu.VMEM((K, EMB), jnp.float32),
                pltpu.SemaphoreType.DMA,
                pltpu.SemaphoreType.DMA,
                pltpu.SemaphoreType.DMA,
                pltpu.SemaphoreType.DMA,
                pltpu.SemaphoreType.DMA,
                pltpu.SemaphoreType.DMA,
            ],
        ))
    return _roll_cache[0]


def kernel(seq_lengths, embeddings):
    seqmat = jnp.broadcast_to(
        jnp.repeat(seq_lengths.astype(jnp.int32), NWORK // BATCH)[:, None],
        (NWORK, 16))
    out = _get_roll()(seqmat, embeddings)
    return out.reshape(BATCH, CONTEXT, EMB)
